# baseline (speedup 1.0000x reference)
import functools

import jax
import jax.numpy as jnp
from jax import lax
from jax.experimental import pallas as pl
from jax.experimental.pallas import tpu as pltpu
from jax.experimental.pallas import tpu_sc as plsc

N_CP = 20
CP_STEP = 40.0
D_PAD = 24
M_COL = 20
K_COL = 21

NC = 2
NS = 16
NW = NC * NS
LANES = 16
IDX_CHUNK = 128


@functools.partial(jax.jit, static_argnames=("b_per_w",))
def _trend_sc(t, idx, fused_tab, b_per_w):
    B = t.shape[0]
    n_chunks = b_per_w // IDX_CHUNK
    n_groups = b_per_w // LANES
    mesh = plsc.VectorSubcoreMesh(core_axis_name="c", subcore_axis_name="s")

    @functools.partial(
        pl.kernel,
        mesh=mesh,
        compiler_params=pltpu.CompilerParams(
            needs_layout_passes=False, use_tc_tiling_on_sc=False
        ),
        out_type=jax.ShapeDtypeStruct((B,), jnp.float32),
        scratch_types=[
            [pltpu.VMEM((IDX_CHUNK,), jnp.int32)] * (b_per_w // IDX_CHUNK),
            pltpu.VMEM((b_per_w,), jnp.float32),
            pltpu.VMEM((b_per_w, D_PAD), jnp.float32),
            pltpu.VMEM((b_per_w,), jnp.float32),
            pltpu.SemaphoreType.DMA,
        ],
    )
    def sc_kernel(t_hbm, idx_hbm, tab_hbm, out_hbm,
                  idx_vs, t_v, rows_v, out_v, sem):
        wid = lax.axis_index("s") * NC + lax.axis_index("c")
        base = wid * b_per_w

        for c in range(n_chunks):
            pltpu.sync_copy(
                idx_hbm.at[pl.ds(base + c * IDX_CHUNK, IDX_CHUNK)], idx_vs[c]
            )

        copies = []
        for c in range(n_chunks):
            sl = pl.ds(c * IDX_CHUNK, IDX_CHUNK)
            copies.append(
                pltpu.async_copy(tab_hbm.at[idx_vs[c]], rows_v.at[sl], sem)
            )
        pltpu.sync_copy(t_hbm.at[pl.ds(base, b_per_w)], t_v)
        for cp in copies:
            cp.wait()

        lane = lax.iota(jnp.int32, LANES)

        def body(g, carry):
            gb = g * LANES
            tg = t_v[pl.ds(gb, LANES)]
            row_ix = gb + lane
            m = plsc.load_gather(rows_v, [row_ix, jnp.full((LANES,), M_COL, jnp.int32)])
            k = plsc.load_gather(rows_v, [row_ix, jnp.full((LANES,), K_COL, jnp.int32)])
            acc = m + k * tg
            for j in range(N_CP):
                col_ix = jnp.full((LANES,), j, jnp.int32)
                d = plsc.load_gather(rows_v, [row_ix, col_ix])
                sj = jnp.float32(CP_STEP * (j + 1))
                acc += jnp.where(tg > sj, d * (tg - sj), 0.0)
            out_v[pl.ds(gb, LANES)] = acc
            return carry

        lax.fori_loop(0, n_groups, body, 0)

        pltpu.sync_copy(out_v, out_hbm.at[pl.ds(base, b_per_w)])

    return sc_kernel(t, idx, fused_tab)


def kernel(t, item_id, m_table, k_table, delta_table):
    B = t.shape[0]
    n_items = delta_table.shape[0]
    fused = jnp.concatenate(
        [
            delta_table,
            m_table,
            k_table,
            jnp.zeros((n_items, D_PAD - N_CP - 2), jnp.float32),
        ],
        axis=1,
    )
    out = _trend_sc(
        t.reshape(B),
        item_id.reshape(B),
        fused,
        b_per_w=B // NW,
    )
    return out.reshape(B, 1)
